# baseline (speedup 1.0000x reference)
import functools

import jax
import jax.numpy as jnp
from jax import lax
from jax.experimental import pallas as pl
from jax.experimental.pallas import tpu as pltpu
from jax.experimental.pallas import tpu_sc as plsc

_B = 4096
_S = 50
_D = 64
_V = 100000
_L = 16

_NC = 2
_NS = 16
_NW = _NC * _NS
_CHB = _B // 2
_NCH = _S * 2
_DPW = _D // _NW
_NBUF = 5
_NROUND = _NCH // _NBUF
_NG = _CHB // _L
_UNROLL = 32

_mesh = plsc.VectorSubcoreMesh(core_axis_name="c", subcore_axis_name="s")


@functools.partial(
    pl.kernel,
    mesh=_mesh,
    out_type=jax.ShapeDtypeStruct((_S, _D, _B), jnp.float32),
    compiler_params=pltpu.CompilerParams(
        use_tc_tiling_on_sc=True, needs_layout_passes=False
    ),
    scratch_types=[
        pltpu.VMEM((1, _V), jnp.float32),
        pltpu.VMEM((_NBUF, 1, _CHB), jnp.int32),
        pltpu.VMEM((_NBUF, 1, _CHB), jnp.float32),
        pltpu.SemaphoreType.DMA((_NBUF,)),
        pltpu.SemaphoreType.DMA((_NBUF,)),
    ],
)
def _sc_lookup(ids_hbm, tab_hbm, out_hbm, row_v, idx_v, obuf, isem, osem):
    wid = lax.axis_index("s") * _NC + lax.axis_index("c")
    d0 = wid * _DPW
    z16 = jnp.zeros((_L,), jnp.int32)

    for r in range(_NBUF):
        pltpu.async_copy(
            ids_hbm.at[pl.ds(r // 2, 1), pl.ds((r % 2) * _CHB, _CHB)],
            idx_v.at[r],
            isem.at[r],
        )

    for dd in range(_DPW):
        d = d0 + dd
        pltpu.sync_copy(tab_hbm.at[pl.ds(d, 1)], row_v)

        def round_(rr, carry):
            c0 = rr * _NBUF
            for r in range(_NBUF):
                ch = c0 + r
                s = ch // 2
                b0 = (ch % 2) * _CHB
                pltpu.make_async_copy(
                    ids_hbm.at[pl.ds(s, 1), pl.ds(b0, _CHB)],
                    idx_v.at[r],
                    isem.at[r],
                ).wait()

                @pl.when(jnp.logical_or(rr > 0, dd > 0))
                def _():
                    pltpu.make_async_copy(
                        obuf.at[r],
                        out_hbm.at[s, pl.ds(d, 1), pl.ds(b0, _CHB)],
                        osem.at[r],
                    ).wait()

                def gather(g, c):
                    base = g * _UNROLL * _L
                    idxs = [
                        idx_v[r, 0, pl.ds(base + u * _L, _L)]
                        for u in range(_UNROLL)
                    ]
                    vals = [
                        plsc.load_gather(row_v, [z16, ix]) for ix in idxs
                    ]
                    for u in range(_UNROLL):
                        obuf[r, 0, pl.ds(base + u * _L, _L)] = vals[u]
                    return c

                lax.fori_loop(0, _NG // _UNROLL, gather, 0)

                pltpu.async_copy(
                    obuf.at[r],
                    out_hbm.at[s, pl.ds(d, 1), pl.ds(b0, _CHB)],
                    osem.at[r],
                )

                c_next = ch + _NBUF
                nxt = jnp.where(c_next < _NCH, c_next, c_next - _NCH)
                sn = nxt // 2
                bn = (nxt % 2) * _CHB

                @pl.when(jnp.logical_or(rr < _NROUND - 1, dd < _DPW - 1))
                def _():
                    pltpu.async_copy(
                        ids_hbm.at[pl.ds(sn, 1), pl.ds(bn, _CHB)],
                        idx_v.at[r],
                        isem.at[r],
                    )

            return carry

        lax.fori_loop(0, _NROUND, round_, 0)

    for r in range(_NBUF):
        ch = _NCH - _NBUF + r
        pltpu.make_async_copy(
            obuf.at[r],
            out_hbm.at[
                ch // 2, pl.ds(d0 + _DPW - 1, 1), pl.ds((ch % 2) * _CHB, _CHB)
            ],
            osem.at[r],
        ).wait()


def kernel(input_ids, embedding_weight, subword_weight):
    ids_t = input_ids.T.astype(jnp.int32)
    tab_t = embedding_weight.T
    out_t = _sc_lookup(ids_t, tab_t)
    return out_t.transpose(2, 0, 1)


# speedup vs baseline: 1.0337x; 1.0337x over previous
import functools

import jax
import jax.numpy as jnp
from jax import lax
from jax.experimental import pallas as pl
from jax.experimental.pallas import tpu as pltpu
from jax.experimental.pallas import tpu_sc as plsc

_B = 4096
_S = 50
_D = 64
_V = 100000
_L = 16

_NC = 2
_NS = 16
_NW = _NC * _NS
_CHB = _B // 2
_NCH = _S * 2
_DPW = _D // _NW
_NBUF = 5
_NROUND = _NCH // _NBUF
_NG = _CHB // _L
_UNROLL = 16

_mesh = plsc.VectorSubcoreMesh(core_axis_name="c", subcore_axis_name="s")


@functools.partial(
    pl.kernel,
    mesh=_mesh,
    out_type=jax.ShapeDtypeStruct((_S, _D, _B), jnp.float32),
    compiler_params=pltpu.CompilerParams(
        use_tc_tiling_on_sc=True, needs_layout_passes=False
    ),
    scratch_types=[
        pltpu.VMEM((1, _V), jnp.float32),
        pltpu.VMEM((_NBUF, 1, _CHB), jnp.int32),
        pltpu.VMEM((_NBUF, 1, _CHB), jnp.float32),
        pltpu.SemaphoreType.DMA((_NBUF,)),
        pltpu.SemaphoreType.DMA((_NBUF,)),
    ],
)
def _sc_lookup(ids_hbm, tab_hbm, out_hbm, row_v, idx_v, obuf, isem, osem):
    wid = lax.axis_index("s") * _NC + lax.axis_index("c")
    d0 = wid * _DPW
    z16 = jnp.zeros((_L,), jnp.int32)

    for r in range(_NBUF):
        pltpu.async_copy(
            ids_hbm.at[pl.ds(r // 2, 1), pl.ds((r % 2) * _CHB, _CHB)],
            idx_v.at[r],
            isem.at[r],
        )

    for dd in range(_DPW):
        d = d0 + dd
        pltpu.sync_copy(tab_hbm.at[pl.ds(d, 1)], row_v)

        def round_(rr, carry):
            c0 = rr * _NBUF
            for r in range(_NBUF):
                ch = c0 + r
                s = ch // 2
                b0 = (ch % 2) * _CHB
                pltpu.make_async_copy(
                    ids_hbm.at[pl.ds(s, 1), pl.ds(b0, _CHB)],
                    idx_v.at[r],
                    isem.at[r],
                ).wait()

                @pl.when(jnp.logical_or(rr > 0, dd > 0))
                def _():
                    pltpu.make_async_copy(
                        obuf.at[r],
                        out_hbm.at[s, pl.ds(d, 1), pl.ds(b0, _CHB)],
                        osem.at[r],
                    ).wait()

                def gather(g, c):
                    base = g * _UNROLL * _L
                    idxs = [
                        idx_v[r, 0, pl.ds(base + u * _L, _L)]
                        for u in range(_UNROLL)
                    ]
                    vals = [
                        plsc.load_gather(row_v, [z16, ix]) for ix in idxs
                    ]
                    for u in range(_UNROLL):
                        obuf[r, 0, pl.ds(base + u * _L, _L)] = vals[u]
                    return c

                lax.fori_loop(0, _NG // _UNROLL, gather, 0)

                pltpu.async_copy(
                    obuf.at[r],
                    out_hbm.at[s, pl.ds(d, 1), pl.ds(b0, _CHB)],
                    osem.at[r],
                )

                c_next = ch + _NBUF
                nxt = jnp.where(c_next < _NCH, c_next, c_next - _NCH)
                sn = nxt // 2
                bn = (nxt % 2) * _CHB

                @pl.when(jnp.logical_or(rr < _NROUND - 1, dd < _DPW - 1))
                def _():
                    pltpu.async_copy(
                        ids_hbm.at[pl.ds(sn, 1), pl.ds(bn, _CHB)],
                        idx_v.at[r],
                        isem.at[r],
                    )

            return carry

        lax.fori_loop(0, _NROUND, round_, 0)

    for r in range(_NBUF):
        ch = _NCH - _NBUF + r
        pltpu.make_async_copy(
            obuf.at[r],
            out_hbm.at[
                ch // 2, pl.ds(d0 + _DPW - 1, 1), pl.ds((ch % 2) * _CHB, _CHB)
            ],
            osem.at[r],
        ).wait()


def kernel(input_ids, embedding_weight, subword_weight):
    ids_t = input_ids.T.astype(jnp.int32)
    tab_t = embedding_weight.T
    out_t = _sc_lookup(ids_t, tab_t)
    return out_t.transpose(2, 0, 1)
